# baseline (speedup 1.0000x reference)
import functools

import jax
import jax.numpy as jnp
from jax import lax
from jax.experimental import pallas as pl
from jax.experimental.pallas import tpu as pltpu
from jax.experimental.pallas import tpu_sc as plsc

EMBED_DIM = 16
BATCH = 16384
CHUNK = 128


def _sc_kernel(uid_hbm, aid_hbm, ut_hbm, at_hbm, wb_hbm, out_hbm,
               uidx_v, aidx_v, urows_v, arows_v, out_v, wb_v,
               sem_u, sem_a):
    nc = lax.axis_size("c")
    wid = lax.axis_index("s") * nc + lax.axis_index("c")
    n_rows_per_worker = uidx_v.shape[0]
    bpw = n_rows_per_worker * CHUNK

    pltpu.sync_copy(uid_hbm.at[pl.ds(wid * n_rows_per_worker, n_rows_per_worker)], uidx_v)
    pltpu.sync_copy(aid_hbm.at[pl.ds(wid * n_rows_per_worker, n_rows_per_worker)], aidx_v)
    pltpu.sync_copy(wb_hbm, wb_v)

    copies = []
    for k in range(n_rows_per_worker):
        copies.append(pltpu.async_copy(
            ut_hbm.at[uidx_v.at[k]], urows_v.at[pl.ds(k * CHUNK, CHUNK)], sem_u))
        copies.append(pltpu.async_copy(
            at_hbm.at[aidx_v.at[k]], arows_v.at[pl.ds(k * CHUNK, CHUNK)], sem_a))
    for c in copies:
        c.wait()

    wu = wb_v[pl.ds(0, EMBED_DIM)]
    wa = wb_v[pl.ds(EMBED_DIM, EMBED_DIM)]
    bias_vec = wb_v[pl.ds(2 * EMBED_DIM, EMBED_DIM)]
    bias = bias_vec[0]
    lanes = lax.iota(jnp.int32, 16)

    def group_body(g, _):
        row_idx = g * 16 + lanes
        acc = jnp.zeros((16,), jnp.float32) + bias
        for d in range(EMBED_DIM):
            col_idx = jnp.full((16,), d, jnp.int32)
            colu = plsc.load_gather(urows_v, [row_idx, col_idx])
            cola = plsc.load_gather(arows_v, [row_idx, col_idx])
            acc = acc + colu * wu[d] + cola * wa[d]
        out_v[pl.ds(g * 16, 16)] = acc
        return 0

    lax.fori_loop(0, bpw // 16, group_body, 0)
    pltpu.sync_copy(out_v, out_hbm.at[pl.ds(wid * bpw, bpw)])


def kernel(user_ids, article_ids, user_table, article_table, fc_w, fc_b):
    info = plsc.get_sparse_core_info()
    nw = info.num_cores * info.num_subcores
    bpw = BATCH // nw
    n_rows_per_worker = bpw // CHUNK

    uid2 = user_ids.reshape(-1, CHUNK).astype(jnp.int32)
    aid2 = article_ids.reshape(-1, CHUNK).astype(jnp.int32)
    wb = jnp.concatenate([fc_w.reshape(-1), fc_b.reshape(-1),
                          jnp.zeros((15,), jnp.float32)])

    mesh = plsc.VectorSubcoreMesh(core_axis_name="c", subcore_axis_name="s")
    out = pl.kernel(
        _sc_kernel,
        mesh=mesh,
        compiler_params=pltpu.CompilerParams(needs_layout_passes=False,
                                             use_tc_tiling_on_sc=False),
        out_type=jax.ShapeDtypeStruct((BATCH,), jnp.float32),
        scratch_types=[
            pltpu.VMEM((n_rows_per_worker, CHUNK), jnp.int32),
            pltpu.VMEM((n_rows_per_worker, CHUNK), jnp.int32),
            pltpu.VMEM((bpw, EMBED_DIM), jnp.float32),
            pltpu.VMEM((bpw, EMBED_DIM), jnp.float32),
            pltpu.VMEM((bpw,), jnp.float32),
            pltpu.VMEM((48,), jnp.float32),
            pltpu.SemaphoreType.DMA,
            pltpu.SemaphoreType.DMA,
        ],
    )(uid2, aid2, user_table, article_table, wb)
    return out.reshape(BATCH, 1)


# speedup vs baseline: 5.9835x; 5.9835x over previous
import jax
import jax.numpy as jnp
from jax import lax
from jax.experimental import pallas as pl
from jax.experimental.pallas import tpu as pltpu
from jax.experimental.pallas import tpu_sc as plsc

EMBED_DIM = 16
BATCH = 16384
GROUP = 16
NGROUP = 32
TCOL = 128
STAGE_COLS = GROUP * TCOL


def _sc_kernel(uid_hbm, aid_hbm, ut_hbm, at_hbm, wb_hbm, out_hbm,
               uidx_v, aidx_v, buf0, buf1, out_v, wb_v, sem0, sem1):
    nc = lax.axis_size("c")
    wid = lax.axis_index("s") * nc + lax.axis_index("c")

    pltpu.sync_copy(uid_hbm.at[pl.ds(wid * 4, 4)], uidx_v)
    pltpu.sync_copy(aid_hbm.at[pl.ds(wid * 4, 4)], aidx_v)
    pltpu.sync_copy(wb_hbm, wb_v)

    bufs = (buf0, buf1)
    sems = (sem0, sem1)
    lanes = lax.iota(jnp.int32, 16)
    wvec = wb_v[pl.ds(0, EMBED_DIM)]
    wvec_a = wb_v[pl.ds(EMBED_DIM, EMBED_DIM)]
    bias = wb_v[pl.ds(2 * EMBED_DIM, EMBED_DIM)][0]

    def run_pass(idx_ref, tbl, ws, first):
        def load_ids(g):
            return idx_ref[g // 8 if isinstance(g, int) else g >> 3,
                           pl.ds((g % 8 if isinstance(g, int) else g & 7) * 16, 16)]

        def fire(g, par):
            idv = load_ids(g)
            for j in range(GROUP):
                cs = (idv[j] >> 7) << 7
                cs = pl.multiple_of(cs, TCOL)
                pltpu.async_copy(tbl.at[:, pl.ds(cs, TCOL)],
                                 bufs[par].at[:, pl.ds(j * TCOL, TCOL)],
                                 sems[par])

        def drain(par):
            pltpu.make_async_copy(tbl.at[:, pl.ds(0, STAGE_COLS)],
                                  bufs[par], sems[par]).wait()

        def compute(g, par):
            idv = load_ids(g)
            colv = lanes * TCOL + (idv & (TCOL - 1))
            sl = pl.ds(g * 16, 16)
            if first:
                acc = jnp.zeros((16,), jnp.float32) + bias
            else:
                acc = out_v[sl]
            for d in range(EMBED_DIM):
                vals = plsc.load_gather(bufs[par], [jnp.full((16,), d, jnp.int32),
                                                    colv])
                acc = acc + vals * ws[d]
            out_v[sl] = acc

        fire(0, 0)
        fire(1, 1)

        def body(k, _):
            g0 = 2 * k
            g1 = 2 * k + 1
            drain(0)
            compute(g0, 0)

            @pl.when(g0 + 2 < NGROUP)
            def _():
                fire(g0 + 2, 0)

            drain(1)
            compute(g1, 1)

            @pl.when(g1 + 2 < NGROUP)
            def _():
                fire(g1 + 2, 1)

            return 0

        lax.fori_loop(0, NGROUP // 2, body, 0)

    wus = [wvec[d] for d in range(EMBED_DIM)]
    was = [wvec_a[d] for d in range(EMBED_DIM)]
    run_pass(uidx_v, ut_hbm, wus, True)
    run_pass(aidx_v, at_hbm, was, False)

    pltpu.sync_copy(out_v, out_hbm.at[pl.ds(wid * 512, 512)])


def kernel(user_ids, article_ids, user_table, article_table, fc_w, fc_b):
    info = plsc.get_sparse_core_info()
    nw = info.num_cores * info.num_subcores
    assert BATCH == nw * 512

    uid2 = user_ids.reshape(-1, 128).astype(jnp.int32)
    aid2 = article_ids.reshape(-1, 128).astype(jnp.int32)
    ut_t = user_table.T
    at_t = article_table.T
    wb = jnp.concatenate([fc_w.reshape(-1), fc_b.reshape(-1),
                          jnp.zeros((15,), jnp.float32)])

    mesh = plsc.VectorSubcoreMesh(core_axis_name="c", subcore_axis_name="s")
    out = pl.kernel(
        _sc_kernel,
        mesh=mesh,
        compiler_params=pltpu.CompilerParams(needs_layout_passes=False),
        out_type=jax.ShapeDtypeStruct((BATCH,), jnp.float32),
        scratch_types=[
            pltpu.VMEM((4, 128), jnp.int32),
            pltpu.VMEM((4, 128), jnp.int32),
            pltpu.VMEM((EMBED_DIM, STAGE_COLS), jnp.float32),
            pltpu.VMEM((EMBED_DIM, STAGE_COLS), jnp.float32),
            pltpu.VMEM((512,), jnp.float32),
            pltpu.VMEM((48,), jnp.float32),
            pltpu.SemaphoreType.DMA,
            pltpu.SemaphoreType.DMA,
        ],
    )(uid2, aid2, ut_t, at_t, wb)
    return out.reshape(BATCH, 1)


# speedup vs baseline: 6.4185x; 1.0727x over previous
import jax
import jax.numpy as jnp
from jax import lax
from jax.experimental import pallas as pl
from jax.experimental.pallas import tpu as pltpu
from jax.experimental.pallas import tpu_sc as plsc

EMBED_DIM = 16
BATCH = 16384
GROUP = 16
NGROUP = 32
TCOL = 128
STAGE_COLS = GROUP * TCOL


NBUF = 3


def _sc_kernel(uid_hbm, aid_hbm, ut_hbm, at_hbm, wb_hbm, out_hbm,
               uidx_v, aidx_v, buf0, buf1, buf2, out_v, wb_v,
               sem0, sem1, sem2):
    nc = lax.axis_size("c")
    wid = lax.axis_index("s") * nc + lax.axis_index("c")

    pltpu.sync_copy(uid_hbm.at[pl.ds(wid * 4, 4)], uidx_v)
    pltpu.sync_copy(aid_hbm.at[pl.ds(wid * 4, 4)], aidx_v)
    pltpu.sync_copy(wb_hbm, wb_v)

    bufs = (buf0, buf1, buf2)
    sems = (sem0, sem1, sem2)
    lanes = lax.iota(jnp.int32, 16)
    wvec = wb_v[pl.ds(0, EMBED_DIM)]
    wvec_a = wb_v[pl.ds(EMBED_DIM, EMBED_DIM)]
    bias = wb_v[pl.ds(2 * EMBED_DIM, EMBED_DIM)][0]

    def run_pass(idx_ref, tbl, ws, first):
        def load_ids(g):
            return idx_ref[g // 8 if isinstance(g, int) else g >> 3,
                           pl.ds((g % 8 if isinstance(g, int) else g & 7) * 16, 16)]

        def fire(g, par):
            idv = load_ids(g)
            for j in range(GROUP):
                cs = (idv[j] >> 7) << 7
                cs = pl.multiple_of(cs, TCOL)
                pltpu.async_copy(tbl.at[:, pl.ds(cs, TCOL)],
                                 bufs[par].at[:, pl.ds(j * TCOL, TCOL)],
                                 sems[par])

        def drain(par):
            pltpu.make_async_copy(tbl.at[:, pl.ds(0, STAGE_COLS)],
                                  bufs[par], sems[par]).wait()

        def compute(g, par):
            idv = load_ids(g)
            colv = lanes * TCOL + (idv & (TCOL - 1))
            sl = pl.ds(g * 16, 16)
            if first:
                acc = jnp.zeros((16,), jnp.float32) + bias
            else:
                acc = out_v[sl]
            for d in range(EMBED_DIM):
                vals = plsc.load_gather(bufs[par], [jnp.full((16,), d, jnp.int32),
                                                    colv])
                acc = acc + vals * ws[d]
            out_v[sl] = acc

        for p in range(NBUF):
            fire(p, p)

        def body(k, _):
            for p in range(NBUF):
                g = NBUF * k + p
                drain(p)
                compute(g, p)

                @pl.when(g + NBUF < NGROUP)
                def _():
                    fire(g + NBUF, p)

            return 0

        lax.fori_loop(0, NGROUP // NBUF, body, 0)
        for g in range(NBUF * (NGROUP // NBUF), NGROUP):
            p = g % NBUF
            drain(p)
            compute(g, p)

    wus = [wvec[d] for d in range(EMBED_DIM)]
    was = [wvec_a[d] for d in range(EMBED_DIM)]
    run_pass(uidx_v, ut_hbm, wus, True)
    run_pass(aidx_v, at_hbm, was, False)

    pltpu.sync_copy(out_v, out_hbm.at[pl.ds(wid * 512, 512)])


def kernel(user_ids, article_ids, user_table, article_table, fc_w, fc_b):
    info = plsc.get_sparse_core_info()
    nw = info.num_cores * info.num_subcores
    assert BATCH == nw * 512

    uid2 = user_ids.reshape(-1, 128).astype(jnp.int32)
    aid2 = article_ids.reshape(-1, 128).astype(jnp.int32)
    ut_t = user_table.T
    at_t = article_table.T
    wb = jnp.concatenate([fc_w.reshape(-1), fc_b.reshape(-1),
                          jnp.zeros((15,), jnp.float32)])

    mesh = plsc.VectorSubcoreMesh(core_axis_name="c", subcore_axis_name="s")
    out = pl.kernel(
        _sc_kernel,
        mesh=mesh,
        compiler_params=pltpu.CompilerParams(needs_layout_passes=False),
        out_type=jax.ShapeDtypeStruct((BATCH,), jnp.float32),
        scratch_types=[
            pltpu.VMEM((4, 128), jnp.int32),
            pltpu.VMEM((4, 128), jnp.int32),
            pltpu.VMEM((EMBED_DIM, STAGE_COLS), jnp.float32),
            pltpu.VMEM((EMBED_DIM, STAGE_COLS), jnp.float32),
            pltpu.VMEM((EMBED_DIM, STAGE_COLS), jnp.float32),
            pltpu.VMEM((512,), jnp.float32),
            pltpu.VMEM((48,), jnp.float32),
            pltpu.SemaphoreType.DMA,
            pltpu.SemaphoreType.DMA,
            pltpu.SemaphoreType.DMA,
        ],
    )(uid2, aid2, ut_t, at_t, wb)
    return out.reshape(BATCH, 1)
